# Initial kernel scaffold; baseline (speedup 1.0000x reference)
import jax
import jax.numpy as jnp
from jax.experimental import pallas as pl


def kernel(indices, table, w0, w1, w2, w3):
    raise NotImplementedError("write your pallas kernel here")


# speedup vs baseline: 1.6105x; 1.6105x over previous
import functools

import jax
import jax.numpy as jnp
from jax import lax
from jax.experimental import pallas as pl
from jax.experimental.pallas import tpu as pltpu
from jax.experimental.pallas import tpu_sc as plsc

B = 1024
L = 200
D = 64
KW = 3
NLAYERS = 4
ROWS = B * L

NC = 2
NS = 16
NW = NC * NS
CH = 128
PER_W = ROWS // NW
CPW = PER_W // CH


def _sc_gather(table, idx3d):
    mesh = plsc.VectorSubcoreMesh(core_axis_name="c", subcore_axis_name="s")

    @functools.partial(
        pl.kernel,
        out_type=jax.ShapeDtypeStruct((ROWS, D), jnp.float32),
        mesh=mesh,
        scratch_types=[
            pltpu.VMEM((CPW, CH), jnp.int32),
            pltpu.VMEM((CH, D), jnp.float32),
            pltpu.VMEM((CH, D), jnp.float32),
            pltpu.SemaphoreType.DMA,
        ],
        compiler_params=pltpu.CompilerParams(use_tc_tiling_on_sc=False),
    )
    def sc_gather(table_hbm, idx_hbm, out_hbm, idx_v, rows0, rows1, gsem):
        wid = lax.axis_index("s") * NC + lax.axis_index("c")
        base_row = wid * PER_W
        pltpu.sync_copy(idx_hbm.at[wid], idx_v)

        def gstart(i, buf):
            pltpu.make_async_copy(table_hbm.at[idx_v.at[i]], buf, gsem).start()

        def gwait(buf):
            pltpu.make_async_copy(table_hbm.at[idx_v.at[0]], buf, gsem).wait()

        def put(i, buf):
            pltpu.sync_copy(buf, out_hbm.at[pl.ds(base_row + i * CH, CH)])

        gstart(0, rows0)

        def body(j, carry):
            i0 = 2 * j
            gwait(rows0)
            gstart(i0 + 1, rows1)
            put(i0, rows0)
            gwait(rows1)

            @pl.when(j + 1 < CPW // 2)
            def _():
                gstart(i0 + 2, rows0)

            put(i0 + 1, rows1)
            return carry

        lax.fori_loop(0, CPW // 2, body, 0)

    return sc_gather(table, idx3d)


def _conv_body(w_ref, x_ref, o_ref, *, nb):
    m = nb * L
    x = x_ref[...].reshape(m, D)
    row = lax.broadcasted_iota(jnp.int32, (m, 1), 0) % L
    not_first = row != 0
    not_last = row != (L - 1)
    zrow = jnp.zeros((1, D), jnp.float32)
    for i in range(NLAYERS):
        xm = jnp.where(not_first, jnp.concatenate([zrow, x[: m - 1, :]], axis=0), 0.0)
        xp = jnp.where(not_last, jnp.concatenate([x[1:, :], zrow], axis=0), 0.0)
        xc = jnp.concatenate([xm, x, xp], axis=1)
        y = lax.dot_general(
            xc, w_ref[i], (((1,), (0,)), ((), ())),
            preferred_element_type=jnp.float32,
        )
        x = jnp.maximum(y, 0.0)
    o_ref[...] = x.reshape(nb, L, D)


def _conv_stack(x, wall, nb=32, interpret=False):
    grid = (B // nb,)
    return pl.pallas_call(
        functools.partial(_conv_body, nb=nb),
        grid=grid,
        in_specs=[
            pl.BlockSpec((NLAYERS, KW * D, D), lambda i: (0, 0, 0)),
            pl.BlockSpec((nb, L, D), lambda i: (i, 0, 0)),
        ],
        out_specs=pl.BlockSpec((nb, L, D), lambda i: (i, 0, 0)),
        out_shape=jax.ShapeDtypeStruct((B, L, D), jnp.float32),
        interpret=interpret,
    )(wall, x)


def kernel(indices, table, w0, w1, w2, w3):
    idx3d = indices.astype(jnp.int32).reshape(NW, CPW, CH)
    gathered = _sc_gather(table, idx3d)
    wall = jnp.stack([w.reshape(KW * D, D) for w in (w0, w1, w2, w3)])
    return _conv_stack(gathered.reshape(B, L, D), wall)
